# baseline (speedup 1.0000x reference)
import functools

import jax
import jax.numpy as jnp
from jax import lax
from jax.experimental import pallas as pl
from jax.experimental.pallas import tpu as pltpu
from jax.experimental.pallas import tpu_sc as plsc

EMB = 128
N_X = 10000
N_EA = 320000
NC, NS = 2, 16
NW = NC * NS

X_PAD = 10240
N_SC_ROWS = 2 * X_PAD
PER_W = N_SC_ROWS // NW
CH = 128
NCH = PER_W // CH
NBUF = 5
N_TAB = 288

TB = 2048
NBTC = -(-N_EA // TB)
EA_PAD = NBTC * TB


def _fire_gather(table, idx_v, c, rows_v, sem):
    pltpu.async_copy(table.at[idx_v.at[pl.ds(c * CH, CH)]], rows_v, sem)


def _wait_gather(table, rows_v, sem):
    pltpu.make_async_copy(table.at[pl.ds(0, CH)], rows_v, sem).wait()


def _fire_out(rows_v, out, wbase, c, sem):
    pltpu.async_copy(rows_v, out.at[pl.ds(wbase + c * CH, CH)], sem)


def _wait_out(rows_v, out, wbase, sem):
    pltpu.make_async_copy(rows_v, out.at[pl.ds(wbase, CH)], sem).wait()


def _sc_body(idx_hbm, table_hbm, out_hbm, idx_v, table_s, *bufs_and_sems):
    rows = bufs_and_sems[:NBUF]
    gsem = bufs_and_sems[NBUF:2 * NBUF]
    osem = bufs_and_sems[2 * NBUF:3 * NBUF]
    sid = lax.axis_index("s")
    wid = sid * NC + lax.axis_index("c")
    wbase = wid * PER_W

    @pl.when(sid == 0)
    def _():
        pltpu.sync_copy(table_hbm, table_s)

    pltpu.sync_copy(idx_hbm.at[pl.ds(wbase, PER_W)], idx_v)
    plsc.subcore_barrier()

    for b in range(NBUF):
        _fire_gather(table_s, idx_v, b, rows[b], gsem[b])

    def step(k, carry):
        c0 = k * NBUF
        for b in range(NBUF):
            _wait_gather(table_s, rows[b], gsem[b])
            _fire_out(rows[b], out_hbm, wbase, c0 + b, osem[b])
        for b in range(NBUF):
            _wait_out(rows[b], out_hbm, wbase, osem[b])
            _fire_gather(table_s, idx_v, c0 + NBUF + b, rows[b], gsem[b])
        return carry

    lax.fori_loop(0, NCH // NBUF - 1, step, 0)

    c0 = NCH - NBUF
    for b in range(NBUF):
        _wait_gather(table_s, rows[b], gsem[b])
        _fire_out(rows[b], out_hbm, wbase, c0 + b, osem[b])
    for b in range(NBUF):
        _wait_out(rows[b], out_hbm, wbase, osem[b])


_sc_gather = functools.partial(
    pl.kernel,
    out_type=jax.ShapeDtypeStruct((N_SC_ROWS, EMB), jnp.float32),
    scratch_types=(
        [pltpu.VMEM((PER_W,), jnp.int32),
         pltpu.VMEM_SHARED((N_TAB, EMB), jnp.float32)]
        + [pltpu.VMEM((CH, EMB), jnp.float32)] * NBUF
        + [pltpu.SemaphoreType.DMA] * (2 * NBUF)
    ),
    mesh=plsc.VectorSubcoreMesh(core_axis_name="c", subcore_axis_name="s"),
    compiler_params=pltpu.CompilerParams(needs_layout_passes=False),
)(_sc_body)


def _tc_body(idx_ref, tab_hi_ref, tab_lo_ref, out_ref):
    idx = idx_ref[0, 0, :]
    oh = (lax.broadcasted_iota(jnp.int32, (TB, 16), 1)
          == idx[:, None]).astype(jnp.bfloat16)
    out_ref[...] = (
        jax.lax.dot(oh, tab_hi_ref[...], preferred_element_type=jnp.float32)
        + jax.lax.dot(oh, tab_lo_ref[...], preferred_element_type=jnp.float32))


_tc_lookup = pl.pallas_call(
    _tc_body,
    grid=(NBTC,),
    in_specs=[
        pl.BlockSpec((1, 1, TB), lambda i: (i, 0, 0)),
        pl.BlockSpec((16, EMB), lambda i: (0, 0)),
        pl.BlockSpec((16, EMB), lambda i: (0, 0)),
    ],
    out_specs=pl.BlockSpec((TB, EMB), lambda i: (i, 0)),
    out_shape=jax.ShapeDtypeStruct((N_EA, EMB), jnp.float32),
)


def kernel(x, edge_attr, tuplefeat, x_table, ea_table, tf_table):
    pair_table = jnp.concatenate(
        [jnp.repeat(tf_table, 16, axis=0), jnp.tile(tf_table, (16, 1))], axis=1)
    sc_table = jnp.concatenate([x_table, pair_table], axis=0)

    tf = tuplefeat.astype(jnp.int32)
    tf_i = tf[:, 0] * 16 + tf[:, 1] + 32
    pad = jnp.zeros((X_PAD - N_X,), jnp.int32)
    sc_idx = jnp.concatenate([
        x.reshape(-1).astype(jnp.int32), pad,
        tf_i, pad,
    ])

    ea_idx = jnp.concatenate([
        edge_attr.astype(jnp.int32),
        jnp.zeros((EA_PAD - N_EA,), jnp.int32),
    ]).reshape(NBTC, 1, TB)

    ea_hi = ea_table.astype(jnp.bfloat16)
    ea_lo = (ea_table - ea_hi.astype(jnp.float32)).astype(jnp.bfloat16)

    sc_out = _sc_gather(sc_idx, sc_table)
    out_ea = _tc_lookup(ea_idx, ea_hi, ea_lo)
    return (sc_out[:N_X],
            out_ea,
            sc_out[X_PAD:X_PAD + N_X])


# speedup vs baseline: 1.6099x; 1.6099x over previous
import functools

import jax
import jax.numpy as jnp
from jax import lax
from jax.experimental import pallas as pl
from jax.experimental.pallas import tpu as pltpu
from jax.experimental.pallas import tpu_sc as plsc

EMB = 128
N_X = 10000
N_EA = 320000
NC, NS = 2, 16
NW = NC * NS

X_PAD = 10240
N_SC_ROWS = 2 * X_PAD
PER_W = N_SC_ROWS // NW
CH = 128
NCH = PER_W // CH
NBUF = 5
N_TAB = 288

TB = 8192
NBTC = -(-N_EA // TB)
EA_PAD = NBTC * TB


def _fire_gather(table, idx_v, c, rows_v, sem):
    pltpu.async_copy(table.at[idx_v.at[pl.ds(c * CH, CH)]], rows_v, sem)


def _wait_gather(table, rows_v, sem):
    pltpu.make_async_copy(table.at[pl.ds(0, CH)], rows_v, sem).wait()


def _fire_out(rows_v, out, wbase, c, sem):
    pltpu.async_copy(rows_v, out.at[pl.ds(wbase + c * CH, CH)], sem)


def _wait_out(rows_v, out, wbase, sem):
    pltpu.make_async_copy(rows_v, out.at[pl.ds(wbase, CH)], sem).wait()


def _sc_body(idx_hbm, table_hbm, out_hbm, idx_v, table_s, *bufs_and_sems):
    rows = bufs_and_sems[:NBUF]
    gsem = bufs_and_sems[NBUF:2 * NBUF]
    osem = bufs_and_sems[2 * NBUF:3 * NBUF]
    sid = lax.axis_index("s")
    wid = sid * NC + lax.axis_index("c")
    wbase = wid * PER_W

    @pl.when(sid == 0)
    def _():
        pltpu.sync_copy(table_hbm, table_s)

    pltpu.sync_copy(idx_hbm.at[pl.ds(wbase, PER_W)], idx_v)
    plsc.subcore_barrier()

    for b in range(NBUF):
        _fire_gather(table_s, idx_v, b, rows[b], gsem[b])

    def step(k, carry):
        c0 = k * NBUF
        for b in range(NBUF):
            _wait_gather(table_s, rows[b], gsem[b])
            _fire_out(rows[b], out_hbm, wbase, c0 + b, osem[b])
        for b in range(NBUF):
            _wait_out(rows[b], out_hbm, wbase, osem[b])
            _fire_gather(table_s, idx_v, c0 + NBUF + b, rows[b], gsem[b])
        return carry

    lax.fori_loop(0, NCH // NBUF - 1, step, 0)

    c0 = NCH - NBUF
    for b in range(NBUF):
        _wait_gather(table_s, rows[b], gsem[b])
        _fire_out(rows[b], out_hbm, wbase, c0 + b, osem[b])
    for b in range(NBUF):
        _wait_out(rows[b], out_hbm, wbase, osem[b])


_sc_gather = functools.partial(
    pl.kernel,
    out_type=jax.ShapeDtypeStruct((N_SC_ROWS, EMB), jnp.float32),
    scratch_types=(
        [pltpu.VMEM((PER_W,), jnp.int32),
         pltpu.VMEM_SHARED((N_TAB, EMB), jnp.float32)]
        + [pltpu.VMEM((CH, EMB), jnp.float32)] * NBUF
        + [pltpu.SemaphoreType.DMA] * (2 * NBUF)
    ),
    mesh=plsc.VectorSubcoreMesh(core_axis_name="c", subcore_axis_name="s"),
    compiler_params=pltpu.CompilerParams(needs_layout_passes=False),
)(_sc_body)


def _tc_body(idx_ref, tab_hi_ref, tab_lo_ref, out_ref):
    idx = idx_ref[0, 0, :]
    oh = (lax.broadcasted_iota(jnp.int32, (TB, 16), 1)
          == idx[:, None]).astype(jnp.bfloat16)
    out_ref[...] = (
        jax.lax.dot(oh, tab_hi_ref[...], preferred_element_type=jnp.float32)
        + jax.lax.dot(oh, tab_lo_ref[...], preferred_element_type=jnp.float32))


_tc_lookup = pl.pallas_call(
    _tc_body,
    grid=(NBTC,),
    in_specs=[
        pl.BlockSpec((1, 1, TB), lambda i: (i, 0, 0)),
        pl.BlockSpec((16, EMB), lambda i: (0, 0)),
        pl.BlockSpec((16, EMB), lambda i: (0, 0)),
    ],
    out_specs=pl.BlockSpec((TB, EMB), lambda i: (i, 0)),
    out_shape=jax.ShapeDtypeStruct((N_EA, EMB), jnp.float32),
)


def kernel(x, edge_attr, tuplefeat, x_table, ea_table, tf_table):
    pair_table = jnp.concatenate(
        [jnp.repeat(tf_table, 16, axis=0), jnp.tile(tf_table, (16, 1))], axis=1)
    sc_table = jnp.concatenate([x_table, pair_table], axis=0)

    tf = tuplefeat.astype(jnp.int32)
    tf_i = tf[:, 0] * 16 + tf[:, 1] + 32
    pad = jnp.zeros((X_PAD - N_X,), jnp.int32)
    sc_idx = jnp.concatenate([
        x.reshape(-1).astype(jnp.int32), pad,
        tf_i, pad,
    ])

    ea_idx = jnp.concatenate([
        edge_attr.astype(jnp.int32),
        jnp.zeros((EA_PAD - N_EA,), jnp.int32),
    ]).reshape(NBTC, 1, TB)

    ea_hi = ea_table.astype(jnp.bfloat16)
    ea_lo = (ea_table - ea_hi.astype(jnp.float32)).astype(jnp.bfloat16)

    sc_out = _sc_gather(sc_idx, sc_table)
    out_ea = _tc_lookup(ea_idx, ea_hi, ea_lo)
    return (sc_out[:N_X],
            out_ea,
            sc_out[X_PAD:X_PAD + N_X])


# speedup vs baseline: 1.8164x; 1.1283x over previous
import functools

import jax
import jax.numpy as jnp
from jax import lax
from jax.experimental import pallas as pl
from jax.experimental.pallas import tpu as pltpu
from jax.experimental.pallas import tpu_sc as plsc

EMB = 128
N_X = 10000
N_EA = 320000
NC, NS = 2, 16
NW = NC * NS

X_PAD = 10240
N_SC_ROWS = 2 * X_PAD
PER_W = N_SC_ROWS // NW
CH = 128
NCH = PER_W // CH
NBUF = 5
N_TAB = 288

TB = 32768
NBTC = -(-N_EA // TB)
EA_PAD = NBTC * TB


def _fire_gather(table, idx_v, c, rows_v, sem):
    pltpu.async_copy(table.at[idx_v.at[pl.ds(c * CH, CH)]], rows_v, sem)


def _wait_gather(table, rows_v, sem):
    pltpu.make_async_copy(table.at[pl.ds(0, CH)], rows_v, sem).wait()


def _fire_out(rows_v, out, wbase, c, sem):
    pltpu.async_copy(rows_v, out.at[pl.ds(wbase + c * CH, CH)], sem)


def _wait_out(rows_v, out, wbase, sem):
    pltpu.make_async_copy(rows_v, out.at[pl.ds(wbase, CH)], sem).wait()


def _sc_body(idx_hbm, table_hbm, out_hbm, idx_v, table_s, *bufs_and_sems):
    rows = bufs_and_sems[:NBUF]
    gsem = bufs_and_sems[NBUF:2 * NBUF]
    osem = bufs_and_sems[2 * NBUF:3 * NBUF]
    sid = lax.axis_index("s")
    wid = sid * NC + lax.axis_index("c")
    wbase = wid * PER_W

    @pl.when(sid == 0)
    def _():
        pltpu.sync_copy(table_hbm, table_s)

    pltpu.sync_copy(idx_hbm.at[pl.ds(wbase, PER_W)], idx_v)
    plsc.subcore_barrier()

    for b in range(NBUF):
        _fire_gather(table_s, idx_v, b, rows[b], gsem[b])

    def step(k, carry):
        c0 = k * NBUF
        for b in range(NBUF):
            _wait_gather(table_s, rows[b], gsem[b])
            _fire_out(rows[b], out_hbm, wbase, c0 + b, osem[b])
        for b in range(NBUF):
            _wait_out(rows[b], out_hbm, wbase, osem[b])
            _fire_gather(table_s, idx_v, c0 + NBUF + b, rows[b], gsem[b])
        return carry

    lax.fori_loop(0, NCH // NBUF - 1, step, 0)

    c0 = NCH - NBUF
    for b in range(NBUF):
        _wait_gather(table_s, rows[b], gsem[b])
        _fire_out(rows[b], out_hbm, wbase, c0 + b, osem[b])
    for b in range(NBUF):
        _wait_out(rows[b], out_hbm, wbase, osem[b])


_sc_gather = functools.partial(
    pl.kernel,
    out_type=jax.ShapeDtypeStruct((N_SC_ROWS, EMB), jnp.float32),
    scratch_types=(
        [pltpu.VMEM((PER_W,), jnp.int32),
         pltpu.VMEM_SHARED((N_TAB, EMB), jnp.float32)]
        + [pltpu.VMEM((CH, EMB), jnp.float32)] * NBUF
        + [pltpu.SemaphoreType.DMA] * (2 * NBUF)
    ),
    mesh=plsc.VectorSubcoreMesh(core_axis_name="c", subcore_axis_name="s"),
    compiler_params=pltpu.CompilerParams(needs_layout_passes=False),
)(_sc_body)


def _tc_body(idx_ref, tab_ref, out_ref):
    idx = idx_ref[0, 0, :]
    oh = ((lax.broadcasted_iota(jnp.int32, (TB, 32), 1) & 15)
          == idx[:, None]).astype(jnp.bfloat16)
    out_ref[...] = jax.lax.dot(oh, tab_ref[...],
                               preferred_element_type=jnp.float32)


_tc_lookup = pl.pallas_call(
    _tc_body,
    grid=(NBTC,),
    in_specs=[
        pl.BlockSpec((1, 1, TB), lambda i: (i, 0, 0)),
        pl.BlockSpec((32, EMB), lambda i: (0, 0)),
    ],
    out_specs=pl.BlockSpec((TB, EMB), lambda i: (i, 0)),
    out_shape=jax.ShapeDtypeStruct((N_EA, EMB), jnp.float32),
)


def kernel(x, edge_attr, tuplefeat, x_table, ea_table, tf_table):
    pair_table = jnp.concatenate(
        [jnp.repeat(tf_table, 16, axis=0), jnp.tile(tf_table, (16, 1))], axis=1)
    sc_table = jnp.concatenate([x_table, pair_table], axis=0)

    tf = tuplefeat.astype(jnp.int32)
    tf_i = tf[:, 0] * 16 + tf[:, 1] + 32
    pad = jnp.zeros((X_PAD - N_X,), jnp.int32)
    sc_idx = jnp.concatenate([
        x.reshape(-1).astype(jnp.int32), pad,
        tf_i, pad,
    ])

    ea_idx = jnp.concatenate([
        edge_attr.astype(jnp.int32),
        jnp.zeros((EA_PAD - N_EA,), jnp.int32),
    ]).reshape(NBTC, 1, TB)

    ea_hi = ea_table.astype(jnp.bfloat16)
    ea_lo = (ea_table - ea_hi.astype(jnp.float32)).astype(jnp.bfloat16)
    ea_hilo = jnp.concatenate([ea_hi, ea_lo], axis=0)

    sc_out = _sc_gather(sc_idx, sc_table)
    out_ea = _tc_lookup(ea_idx, ea_hilo)
    return (sc_out[:N_X],
            out_ea,
            sc_out[X_PAD:X_PAD + N_X])


# speedup vs baseline: 1.8201x; 1.0020x over previous
import functools

import jax
import jax.numpy as jnp
from jax import lax
from jax.experimental import pallas as pl
from jax.experimental.pallas import tpu as pltpu
from jax.experimental.pallas import tpu_sc as plsc

EMB = 128
N_X = 10000
N_EA = 320000
NC, NS = 2, 16
NW = NC * NS

X_PAD = 10240
N_SC_ROWS = 2 * X_PAD
PER_W = N_SC_ROWS // NW
CH = 128
NCH = PER_W // CH
NBUF = 5
N_TAB = 288

TB = 32768
NBTC = -(-N_EA // TB)
EA_PAD = NBTC * TB


def _fire_gather(table, idx_v, c, rows_v, sem):
    pltpu.async_copy(table.at[idx_v.at[pl.ds(c * CH, CH)]], rows_v, sem)


def _wait_gather(table, rows_v, sem):
    pltpu.make_async_copy(table.at[pl.ds(0, CH)], rows_v, sem).wait()


def _fire_out(rows_v, out, wbase, c, sem):
    pltpu.async_copy(rows_v, out.at[pl.ds(wbase + c * CH, CH)], sem)


def _wait_out(rows_v, out, wbase, sem):
    pltpu.make_async_copy(rows_v, out.at[pl.ds(wbase, CH)], sem).wait()


def _sc_body(idx_hbm, table_hbm, out_hbm, idx_v, table_s, *bufs_and_sems):
    rows = bufs_and_sems[:NBUF]
    gsem = bufs_and_sems[NBUF:2 * NBUF]
    osem = bufs_and_sems[2 * NBUF:3 * NBUF]
    sid = lax.axis_index("s")
    wid = sid * NC + lax.axis_index("c")
    wbase = wid * PER_W

    @pl.when(sid == 0)
    def _():
        pltpu.sync_copy(table_hbm, table_s)

    pltpu.sync_copy(idx_hbm.at[pl.ds(wbase, PER_W)], idx_v)
    plsc.subcore_barrier()

    for b in range(NBUF):
        _fire_gather(table_s, idx_v, b, rows[b], gsem[b])

    def step(k, carry):
        c0 = k * NBUF
        for b in range(NBUF):
            _wait_gather(table_s, rows[b], gsem[b])
            _fire_out(rows[b], out_hbm, wbase, c0 + b, osem[b])
        for b in range(NBUF):
            _wait_out(rows[b], out_hbm, wbase, osem[b])
            _fire_gather(table_s, idx_v, c0 + NBUF + b, rows[b], gsem[b])
        return carry

    lax.fori_loop(0, NCH // NBUF - 1, step, 0)

    c0 = NCH - NBUF
    for b in range(NBUF):
        _wait_gather(table_s, rows[b], gsem[b])
        _fire_out(rows[b], out_hbm, wbase, c0 + b, osem[b])
    for b in range(NBUF):
        _wait_out(rows[b], out_hbm, wbase, osem[b])


_sc_gather = functools.partial(
    pl.kernel,
    out_type=jax.ShapeDtypeStruct((N_SC_ROWS, EMB), jnp.float32),
    scratch_types=(
        [pltpu.VMEM((PER_W,), jnp.int32),
         pltpu.VMEM_SHARED((N_TAB, EMB), jnp.float32)]
        + [pltpu.VMEM((CH, EMB), jnp.float32)] * NBUF
        + [pltpu.SemaphoreType.DMA] * (2 * NBUF)
    ),
    mesh=plsc.VectorSubcoreMesh(core_axis_name="c", subcore_axis_name="s"),
    compiler_params=pltpu.CompilerParams(needs_layout_passes=False),
)(_sc_body)


def _tc_body(idx_ref, tab_ref, out_ref):
    idx = idx_ref[0, 0, :]
    oh = ((lax.broadcasted_iota(jnp.int32, (TB, 32), 1) & 15)
          == idx[:, None]).astype(jnp.bfloat16)
    out_ref[...] = jax.lax.dot(oh, tab_ref[...],
                               preferred_element_type=jnp.float32)


_tc_lookup = pl.pallas_call(
    _tc_body,
    grid=(NBTC,),
    in_specs=[
        pl.BlockSpec((1, 1, TB), lambda i: (i, 0, 0)),
        pl.BlockSpec((32, EMB), lambda i: (0, 0)),
    ],
    out_specs=pl.BlockSpec((TB, EMB), lambda i: (i, 0)),
    out_shape=jax.ShapeDtypeStruct((N_EA, EMB), jnp.float32),
)


def kernel(x, edge_attr, tuplefeat, x_table, ea_table, tf_table):
    pair_table = jnp.concatenate(
        [jnp.repeat(tf_table, 16, axis=0), jnp.tile(tf_table, (16, 1))], axis=1)
    sc_table = jnp.concatenate([x_table, pair_table], axis=0)

    tf = tuplefeat.astype(jnp.int32)
    tf_i = tf[:, 0] * 16 + tf[:, 1] + 32
    pad = jnp.zeros((X_PAD - N_X,), jnp.int32)
    sc_idx = jnp.concatenate([
        x.reshape(-1).astype(jnp.int32), pad,
        tf_i, pad,
    ])

    ea_idx = jnp.concatenate([
        edge_attr.astype(jnp.int32),
        jnp.zeros((EA_PAD - N_EA,), jnp.int32),
    ]).reshape(NBTC, 1, TB)

    ea_hi = ea_table.astype(jnp.bfloat16)
    ea_lo = (ea_table - ea_hi.astype(jnp.float32)).astype(jnp.bfloat16)
    ea_hilo = jnp.concatenate([ea_hi, ea_lo], axis=0)

    sc_out = _sc_gather(sc_idx, sc_table)
    out_ea = _tc_lookup(ea_idx, ea_hilo)
    sc_out, out_ea = lax.optimization_barrier((sc_out, out_ea))
    return (sc_out[:N_X],
            out_ea,
            sc_out[X_PAD:X_PAD + N_X])
